# baseline (speedup 1.0000x reference)
import functools

import jax
import jax.numpy as jnp
from jax import lax
from jax.experimental import pallas as pl
from jax.experimental.pallas import tpu as pltpu
from jax.experimental.pallas import tpu_sc as plsc

NC = 2
NS = 16
NW = NC * NS


def _matmul2(x, w_stack, V, D, H, bv):
    nb = V // bv

    def body(x_ref, w_ref, o_ref):
        o_ref[...] = jnp.dot(x_ref[...], w_ref[0],
                             preferred_element_type=jnp.float32)

    return pl.pallas_call(
        body,
        grid=(2, nb),
        in_specs=[
            pl.BlockSpec((bv, D), lambda t, i: (i, 0)),
            pl.BlockSpec((1, D, H), lambda t, i: (t, 0, 0)),
        ],
        out_specs=pl.BlockSpec((bv, H), lambda t, i, _nb=nb: (t * _nb + i, 0)),
        out_shape=jax.ShapeDtypeStruct((2 * V, H), jnp.float32),
    )(x, w_stack)


def _sc_segment_sum(y, src, tgt, Vp, H, K, nch):
    ept = nch * K
    rpt = Vp // NS
    zr = 128
    nz = rpt // zr
    zrem = rpt - nz * zr
    npair = nch // 2

    mesh = plsc.VectorSubcoreMesh(core_axis_name="c", subcore_axis_name="s",
                                  num_cores=NC, num_subcores=NS)

    @functools.partial(
        pl.kernel,
        out_type=jax.ShapeDtypeStruct((NC * Vp, H), jnp.float32),
        mesh=mesh,
        scratch_types=[
            [pltpu.VMEM((K,), jnp.int32)] * 2,
            [pltpu.VMEM((K,), jnp.int32)] * 2,
            [pltpu.VMEM((K, H), jnp.float32)] * 2,
            pltpu.VMEM((zr, H), jnp.float32),
            pltpu.VMEM_SHARED((Vp, H), jnp.float32),
            [pltpu.SemaphoreType.DMA] * 2,
            [pltpu.SemaphoreType.DMA] * 2,
            [pltpu.SemaphoreType.DMA] * 2,
        ],
    )
    def body(y_hbm, src_hbm, tgt_hbm, out_hbm,
             sidx, tidx, rows, zbuf, acc, isem, gsem, ssem):
        c = lax.axis_index("c")
        s = lax.axis_index("s")
        wid = s * NC + c
        ebase = wid * ept

        def idx_start(b, i):
            base = ebase + i * K
            pltpu.make_async_copy(
                src_hbm.at[pl.ds(base, K)], sidx[b], isem[b]).start()
            pltpu.make_async_copy(
                tgt_hbm.at[pl.ds(base, K)], tidx[b], isem[b]).start()

        def idx_wait(b):
            pltpu.make_async_copy(
                src_hbm.at[pl.ds(0, K)], sidx[b], isem[b]).wait()
            pltpu.make_async_copy(
                tgt_hbm.at[pl.ds(0, K)], tidx[b], isem[b]).wait()

        def gather_start(b):
            pltpu.make_async_copy(y_hbm.at[sidx[b]], rows[b], gsem[b]).start()

        def gather_wait(b):
            pltpu.make_async_copy(y_hbm.at[sidx[b]], rows[b], gsem[b]).wait()

        def scat_start(b):
            pltpu.make_async_copy(
                rows[b], acc.at[tidx[b]], ssem[b]).start(add=True)

        def scat_wait(b):
            pltpu.make_async_copy(rows[b], acc.at[tidx[b]], ssem[b]).wait()

        idx_start(0, 0)
        idx_start(1, 1)

        def zrow(r, t):
            for j in range(H // 16):
                zbuf[r, pl.ds(j * 16, 16)] = jnp.zeros((16,), jnp.float32)
            return t
        lax.fori_loop(0, zr, zrow, 0)
        for k in range(nz):
            pltpu.sync_copy(zbuf, acc.at[pl.ds(s * rpt + k * zr, zr)])
        if zrem:
            pltpu.sync_copy(zbuf.at[pl.ds(0, zrem)],
                            acc.at[pl.ds(s * rpt + nz * zr, zrem)])

        idx_wait(0)
        idx_wait(1)
        plsc.subcore_barrier()

        def step(i, t):
            idx_start(0, i)
            idx_wait(0)
            gather_start(0)
            gather_wait(0)
            scat_start(0)
            scat_wait(0)
            return t
        lax.fori_loop(0, nch, step, 0)
        plsc.subcore_barrier()

        pltpu.sync_copy(acc.at[pl.ds(s * rpt, rpt)],
                        out_hbm.at[pl.ds(c * Vp + s * rpt, rpt)])

    return body(y, src, tgt)


def _add_relu(p, V, H, bv):
    def body(p_ref, o_ref):
        o_ref[...] = jnp.maximum(p_ref[0] + p_ref[1], 0.0)

    return pl.pallas_call(
        body,
        grid=(V // bv,),
        in_specs=[pl.BlockSpec((2, bv, H), lambda i: (0, i, 0))],
        out_specs=pl.BlockSpec((bv, H), lambda i: (i, 0)),
        out_shape=jax.ShapeDtypeStruct((V, H), jnp.float32),
    )(p)


def kernel(node_embeddings, adjacency_list_0, adjacency_list_1, W0, W1):
    V, D = node_embeddings.shape
    H = W0.shape[1]
    E = adjacency_list_0.shape[0]

    Vp = 10112
    K = 128
    ET = 2 * E
    nch = -(-ET // (NW * K))
    nch += nch % 2
    ETp = NW * K * nch
    npad = ETp - ET
    pad_src = jnp.zeros((npad + 2 * K,), jnp.int32)
    pad_tgt = jnp.concatenate([
        V + (jnp.arange(npad, dtype=jnp.int32) % (Vp - V)),
        jnp.zeros((2 * K,), jnp.int32),
    ])

    src = jnp.concatenate(
        [adjacency_list_0[:, 0], adjacency_list_1[:, 0] + V, pad_src])
    tgt = jnp.concatenate(
        [adjacency_list_0[:, 1], adjacency_list_1[:, 1], pad_tgt])
    w_stack = jnp.stack([W0, W1])

    y = _matmul2(node_embeddings, w_stack, V, D, H, bv=2000)
    partials = _sc_segment_sum(y, src, tgt, Vp, H, K, nch)
    return _add_relu(partials.reshape(NC, Vp, H), V, H, bv=2000)


# speedup vs baseline: 1.2423x; 1.2423x over previous
import functools

import jax
import jax.numpy as jnp
from jax import lax
from jax.experimental import pallas as pl
from jax.experimental.pallas import tpu as pltpu
from jax.experimental.pallas import tpu_sc as plsc

NC = 2
NS = 16
NW = NC * NS


def _matmul2(x, w_stack, V, D, H, bv):
    nb = V // bv

    def body(x_ref, w_ref, o_ref):
        o_ref[...] = jnp.dot(x_ref[...], w_ref[0],
                             preferred_element_type=jnp.float32)

    return pl.pallas_call(
        body,
        grid=(2, nb),
        in_specs=[
            pl.BlockSpec((bv, D), lambda t, i: (i, 0)),
            pl.BlockSpec((1, D, H), lambda t, i: (t, 0, 0)),
        ],
        out_specs=pl.BlockSpec((bv, H), lambda t, i, _nb=nb: (t * _nb + i, 0)),
        out_shape=jax.ShapeDtypeStruct((2 * V, H), jnp.float32),
    )(x, w_stack)


def _sc_segment_sum(y, src, tgt, Vp, H, K, nch):
    ept = nch * K
    rpt = Vp // NS
    zr = 128
    nz = rpt // zr
    zrem = rpt - nz * zr
    npair = nch // 2

    mesh = plsc.VectorSubcoreMesh(core_axis_name="c", subcore_axis_name="s",
                                  num_cores=NC, num_subcores=NS)

    @functools.partial(
        pl.kernel,
        out_type=jax.ShapeDtypeStruct((NC * Vp, H), jnp.float32),
        mesh=mesh,
        scratch_types=[
            [pltpu.VMEM((K,), jnp.int32)] * 2,
            [pltpu.VMEM((K,), jnp.int32)] * 2,
            [pltpu.VMEM((K, H), jnp.float32)] * 2,
            pltpu.VMEM((zr, H), jnp.float32),
            pltpu.VMEM_SHARED((Vp, H), jnp.float32),
            [pltpu.SemaphoreType.DMA] * 2,
            [pltpu.SemaphoreType.DMA] * 2,
            [pltpu.SemaphoreType.DMA] * 2,
        ],
    )
    def body(y_hbm, src_hbm, tgt_hbm, out_hbm,
             sidx, tidx, rows, zbuf, acc, isem, gsem, ssem):
        c = lax.axis_index("c")
        s = lax.axis_index("s")
        wid = s * NC + c
        ebase = wid * ept

        def idx_start(b, i):
            base = ebase + i * K
            pltpu.make_async_copy(
                src_hbm.at[pl.ds(base, K)], sidx[b], isem[b]).start()
            pltpu.make_async_copy(
                tgt_hbm.at[pl.ds(base, K)], tidx[b], isem[b]).start()

        def idx_wait(b):
            pltpu.make_async_copy(
                src_hbm.at[pl.ds(0, K)], sidx[b], isem[b]).wait()
            pltpu.make_async_copy(
                tgt_hbm.at[pl.ds(0, K)], tidx[b], isem[b]).wait()

        def gather_start(b):
            pltpu.make_async_copy(y_hbm.at[sidx[b]], rows[b], gsem[b]).start()

        def gather_wait(b):
            pltpu.make_async_copy(y_hbm.at[sidx[b]], rows[b], gsem[b]).wait()

        def scat_start(b):
            pltpu.make_async_copy(
                rows[b], acc.at[tidx[b]], ssem[b]).start(add=True)

        def scat_wait(b):
            pltpu.make_async_copy(rows[b], acc.at[tidx[b]], ssem[b]).wait()

        idx_start(0, 0)
        idx_start(1, 1)

        def zrow(r, t):
            for j in range(H // 16):
                zbuf[r, pl.ds(j * 16, 16)] = jnp.zeros((16,), jnp.float32)
            return t
        lax.fori_loop(0, zr, zrow, 0)
        for k in range(nz):
            pltpu.sync_copy(zbuf, acc.at[pl.ds(s * rpt + k * zr, zr)])
        if zrem:
            pltpu.sync_copy(zbuf.at[pl.ds(0, zrem)],
                            acc.at[pl.ds(s * rpt + nz * zr, zrem)])

        idx_wait(0)
        idx_wait(1)
        plsc.subcore_barrier()

        def step(i, t):
            idx_start(0, i)
            idx_wait(0)
            gather_start(0)
            gather_wait(0)
            scat_start(0)
            scat_wait(0)
            return t
        lax.fori_loop(0, nch, step, 0)
        plsc.subcore_barrier()

        pltpu.sync_copy(acc.at[pl.ds(s * rpt, rpt)],
                        out_hbm.at[pl.ds(c * Vp + s * rpt, rpt)])

    return body(y, src, tgt)


def _add_relu(p, V, H, bv):
    def body(p_ref, o_ref):
        o_ref[...] = jnp.maximum(p_ref[0] + p_ref[1], 0.0)

    return pl.pallas_call(
        body,
        grid=(V // bv,),
        in_specs=[pl.BlockSpec((2, bv, H), lambda i: (0, i, 0))],
        out_specs=pl.BlockSpec((bv, H), lambda i: (i, 0)),
        out_shape=jax.ShapeDtypeStruct((V, H), jnp.float32),
    )(p)


def kernel(node_embeddings, adjacency_list_0, adjacency_list_1, W0, W1):
    V, D = node_embeddings.shape
    H = W0.shape[1]
    E = adjacency_list_0.shape[0]

    Vp = 10112
    K = 64
    ET = 2 * E
    nch = -(-ET // (NW * K))
    nch += nch % 2
    ETp = NW * K * nch
    npad = ETp - ET
    pad_src = jnp.zeros((npad + 2 * K,), jnp.int32)
    pad_tgt = jnp.concatenate([
        V + (jnp.arange(npad, dtype=jnp.int32) % (Vp - V)),
        jnp.zeros((2 * K,), jnp.int32),
    ])

    src = jnp.concatenate(
        [adjacency_list_0[:, 0], adjacency_list_1[:, 0] + V, pad_src])
    tgt = jnp.concatenate(
        [adjacency_list_0[:, 1], adjacency_list_1[:, 1], pad_tgt])
    w_stack = jnp.stack([W0, W1])

    y = _matmul2(node_embeddings, w_stack, V, D, H, bv=2000)
    partials = _sc_segment_sum(y, src, tgt, Vp, H, K, nch)
    return _add_relu(partials.reshape(NC, Vp, H), V, H, bv=2000)


# speedup vs baseline: 1.7068x; 1.3739x over previous
import functools

import jax
import jax.numpy as jnp
from jax import lax
from jax.experimental import pallas as pl
from jax.experimental.pallas import tpu as pltpu
from jax.experimental.pallas import tpu_sc as plsc

NC = 2
NS = 16
NW = NC * NS


def _matmul2(x, w_stack, V, D, H, bv):
    nb = V // bv

    def body(x_ref, w_ref, o_ref):
        o_ref[...] = jnp.dot(x_ref[...], w_ref[0],
                             preferred_element_type=jnp.float32)

    return pl.pallas_call(
        body,
        grid=(2, nb),
        in_specs=[
            pl.BlockSpec((bv, D), lambda t, i: (i, 0)),
            pl.BlockSpec((1, D, H), lambda t, i: (t, 0, 0)),
        ],
        out_specs=pl.BlockSpec((bv, H), lambda t, i, _nb=nb: (t * _nb + i, 0)),
        out_shape=jax.ShapeDtypeStruct((2 * V, H), jnp.float32),
    )(x, w_stack)


def _sc_segment_sum(y, src, tgt, Vp, H, K):
    ET = src.shape[0]
    ept = ET // NW
    nch = ept // K
    rpt = Vp // NS
    zr = 128
    nz = rpt // zr

    mesh = plsc.VectorSubcoreMesh(core_axis_name="c", subcore_axis_name="s",
                                  num_cores=NC, num_subcores=NS)

    @functools.partial(
        pl.kernel,
        out_type=jax.ShapeDtypeStruct((NC * Vp, H), jnp.float32),
        mesh=mesh,
        scratch_types=[
            pltpu.VMEM((K,), jnp.int32),
            pltpu.VMEM((K,), jnp.int32),
            pltpu.VMEM((K, H), jnp.float32),
            pltpu.VMEM((zr, H), jnp.float32),
            pltpu.VMEM_SHARED((Vp, H), jnp.float32),
            pltpu.SemaphoreType.DMA,
        ],
    )
    def body(y_hbm, src_hbm, tgt_hbm, out_hbm, sidx, tidx, rows, zbuf, acc, sem):
        c = lax.axis_index("c")
        s = lax.axis_index("s")
        wid = s * NC + c

        def zrow(r, t):
            for j in range(H // 16):
                zbuf[r, pl.ds(j * 16, 16)] = jnp.zeros((16,), jnp.float32)
            return t
        lax.fori_loop(0, zr, zrow, 0)
        for k in range(nz):
            pltpu.sync_copy(zbuf, acc.at[pl.ds(s * rpt + k * zr, zr)])
        plsc.subcore_barrier()

        def step(i, t):
            base = wid * ept + i * K
            pltpu.sync_copy(src_hbm.at[pl.ds(base, K)], sidx)
            pltpu.sync_copy(tgt_hbm.at[pl.ds(base, K)], tidx)
            pltpu.async_copy(y_hbm.at[sidx], rows, sem).wait()
            pltpu.sync_copy(rows, acc.at[tidx], add=True)
            return t
        lax.fori_loop(0, nch, step, 0)
        plsc.subcore_barrier()

        pltpu.sync_copy(acc.at[pl.ds(s * rpt, rpt)],
                        out_hbm.at[pl.ds(c * Vp + s * rpt, rpt)])

    return body(y, src, tgt)


def _add_relu(p, V, H, bv):
    def body(p_ref, o_ref):
        o_ref[...] = jnp.maximum(p_ref[0] + p_ref[1], 0.0)

    return pl.pallas_call(
        body,
        grid=(V // bv,),
        in_specs=[pl.BlockSpec((2, bv, H), lambda i: (0, i, 0))],
        out_specs=pl.BlockSpec((bv, H), lambda i: (i, 0)),
        out_shape=jax.ShapeDtypeStruct((V, H), jnp.float32),
    )(p)


def kernel(node_embeddings, adjacency_list_0, adjacency_list_1, W0, W1):
    V, D = node_embeddings.shape
    H = W0.shape[1]

    src = jnp.concatenate([adjacency_list_0[:, 0], adjacency_list_1[:, 0] + V])
    tgt = jnp.concatenate([adjacency_list_0[:, 1], adjacency_list_1[:, 1]])
    w_stack = jnp.stack([W0, W1])

    Vp = 10240
    y = _matmul2(node_embeddings, w_stack, V, D, H, bv=2000)
    partials = _sc_segment_sum(y, src, tgt, Vp, H, K=80)
    return _add_relu(partials.reshape(NC, Vp, H), V, H, bv=2000)


# speedup vs baseline: 1.9813x; 1.1609x over previous
import functools

import jax
import jax.numpy as jnp
from jax import lax
from jax.experimental import pallas as pl
from jax.experimental.pallas import tpu as pltpu
from jax.experimental.pallas import tpu_sc as plsc

NC = 2
NS = 16
NW = NC * NS


def _matmul2(x, w_stack, V, D, H, bv):
    nb = V // bv

    def body(x_ref, w_ref, o_ref):
        o_ref[...] = jnp.dot(x_ref[...], w_ref[0],
                             preferred_element_type=jnp.float32)

    return pl.pallas_call(
        body,
        grid=(2, nb),
        in_specs=[
            pl.BlockSpec((bv, D), lambda t, i: (i, 0)),
            pl.BlockSpec((1, D, H), lambda t, i: (t, 0, 0)),
        ],
        out_specs=pl.BlockSpec((bv, H), lambda t, i, _nb=nb: (t * _nb + i, 0)),
        out_shape=jax.ShapeDtypeStruct((2 * V, H), jnp.float32),
    )(x, w_stack)


def _sc_segment_sum(y, src, tgt, Vp, H, K):
    ET = src.shape[0]
    ept = ET // NW
    nch = ept // K
    rpt = Vp // NS
    zr = 128
    nz = rpt // zr

    mesh = plsc.VectorSubcoreMesh(core_axis_name="c", subcore_axis_name="s",
                                  num_cores=NC, num_subcores=NS)

    @functools.partial(
        pl.kernel,
        out_type=jax.ShapeDtypeStruct((NC * Vp, H), jnp.float32),
        mesh=mesh,
        scratch_types=[
            pltpu.VMEM((K,), jnp.int32),
            pltpu.VMEM((K,), jnp.int32),
            pltpu.VMEM((K, H), jnp.float32),
            pltpu.VMEM((zr, H), jnp.float32),
            pltpu.VMEM_SHARED((Vp, H), jnp.float32),
            [pltpu.SemaphoreType.DMA] * 3,
        ],
    )
    def body(y_hbm, src_hbm, tgt_hbm, out_hbm, sidx, tidx, rows, zbuf, acc, sems):
        c = lax.axis_index("c")
        s = lax.axis_index("s")
        wid = s * NC + c

        def zrow(r, t):
            for j in range(H // 16):
                zbuf[r, pl.ds(j * 16, 16)] = jnp.zeros((16,), jnp.float32)
            return t
        lax.fori_loop(0, zr, zrow, 0)
        for k in range(nz):
            pltpu.sync_copy(zbuf, acc.at[pl.ds(s * rpt + k * zr, zr)])
        plsc.subcore_barrier()

        isem, gsem, ssem = sems

        def step(i, t):
            base = wid * ept + i * K
            pltpu.make_async_copy(
                src_hbm.at[pl.ds(base, K)], sidx, isem).start()
            pltpu.make_async_copy(
                tgt_hbm.at[pl.ds(base, K)], tidx, isem).start()
            pltpu.make_async_copy(
                src_hbm.at[pl.ds(base, K)], sidx, isem).wait()
            pltpu.make_async_copy(
                tgt_hbm.at[pl.ds(base, K)], tidx, isem).wait()
            pltpu.make_async_copy(y_hbm.at[sidx], rows, gsem).start()
            pltpu.make_async_copy(y_hbm.at[sidx], rows, gsem).wait()
            pltpu.make_async_copy(rows, acc.at[tidx], ssem).start(add=True)
            pltpu.make_async_copy(rows, acc.at[tidx], ssem).wait()
            return t
        lax.fori_loop(0, nch, step, 0)
        plsc.subcore_barrier()

        pltpu.sync_copy(acc.at[pl.ds(s * rpt, rpt)],
                        out_hbm.at[pl.ds(c * Vp + s * rpt, rpt)])

    return body(y, src, tgt)


def _add_relu(p, V, H, bv):
    def body(p_ref, o_ref):
        o_ref[...] = jnp.maximum(p_ref[0] + p_ref[1], 0.0)

    return pl.pallas_call(
        body,
        grid=(V // bv,),
        in_specs=[pl.BlockSpec((2, bv, H), lambda i: (0, i, 0))],
        out_specs=pl.BlockSpec((bv, H), lambda i: (i, 0)),
        out_shape=jax.ShapeDtypeStruct((V, H), jnp.float32),
    )(p)


def kernel(node_embeddings, adjacency_list_0, adjacency_list_1, W0, W1):
    V, D = node_embeddings.shape
    H = W0.shape[1]

    src = jnp.concatenate([adjacency_list_0[:, 0], adjacency_list_1[:, 0] + V])
    tgt = jnp.concatenate([adjacency_list_0[:, 1], adjacency_list_1[:, 1]])
    w_stack = jnp.stack([W0, W1])

    Vp = 10240
    y = _matmul2(node_embeddings, w_stack, V, D, H, bv=2000)
    partials = _sc_segment_sum(y, src, tgt, Vp, H, K=80)
    return _add_relu(partials.reshape(NC, Vp, H), V, H, bv=2000)
